# baseline (speedup 1.0000x reference)
import functools

import jax
import jax.numpy as jnp
from jax import lax
from jax.experimental import pallas as pl
from jax.experimental.pallas import tpu as pltpu
from jax.experimental.pallas import tpu_sc as plsc


def kernel(x):
    b, s, m, d = x.shape
    cr = m // 2
    nc = cr - 2
    P = b * s
    NW = 32
    per_w = P // NW
    CH = 4
    n_chunks = per_w // CH
    L = 16

    xf = x.reshape(P, m, d)

    mesh = plsc.VectorSubcoreMesh(core_axis_name="c", subcore_axis_name="s")

    @functools.partial(
        pl.kernel,
        mesh=mesh,
        out_type=jax.ShapeDtypeStruct((P, nc, 6, d), jnp.float32),
        scratch_types=[
            pltpu.VMEM((CH, m, d), jnp.float32),
            pltpu.VMEM((CH, nc, 6, d), jnp.float32),
        ],
    )
    def recombine(x_hbm, out_hbm, in_v, out_v):
        cid = lax.axis_index("c")
        sid = lax.axis_index("s")
        wid = sid * 2 + cid
        base = wid * per_w

        def body(i, carry):
            p0 = base + i * CH
            pltpu.sync_copy(x_hbm.at[pl.ds(p0, CH)], in_v)
            for p in range(CH):
                halves = {}

                def get(r, h, _p=p):
                    key = (r, h)
                    if key not in halves:
                        halves[key] = in_v[_p, r, pl.ds(h * L, L)]
                    return halves[key]

                for c in range(nc):
                    rows = (0, 1, 2 + c, cr, cr + 1, cr + 2 + c)
                    for j, r in enumerate(rows):
                        for h in range(d // L):
                            out_v[p, c, j, pl.ds(h * L, L)] = get(r, h)
            pltpu.sync_copy(out_v, out_hbm.at[pl.ds(p0, CH)])
            return carry

        lax.fori_loop(0, n_chunks, body, 0)

    out = recombine(xf)
    return out.reshape(b, s, nc, 6, d)
